# Initial kernel scaffold; baseline (speedup 1.0000x reference)
import jax
import jax.numpy as jnp
from jax.experimental import pallas as pl


def kernel(coeffs, weight):
    raise NotImplementedError("write your pallas kernel here")


# speedup vs baseline: 5.2562x; 5.2562x over previous
import functools
import math

import jax
import jax.numpy as jnp
from jax import lax
from jax.experimental import pallas as pl
from jax.experimental.pallas import tpu as pltpu
from jax.experimental.pallas import tpu_sc as plsc

_P = 1000
_D = 16
_MAX_NORM = math.sqrt(_D)

_NC = 2
_NS = 16
_NW = _NC * _NS


def _renorm_body(w_ref, o_ref):
    w = w_ref[...]
    norm = jnp.sqrt(jnp.sum(w * w, axis=1, keepdims=True))
    scale = jnp.where(norm > _MAX_NORM, _MAX_NORM / (norm + 1e-7), 1.0)
    o_ref[...] = w * scale


def _renorm_table(weight):
    return pl.pallas_call(
        _renorm_body,
        out_shape=jax.ShapeDtypeStruct(weight.shape, weight.dtype),
    )(weight)


def _make_gather(n_rows, chunk):
    n_chunks = n_rows // (_NW * chunk)
    b_per_w = n_rows // _NW
    mesh = plsc.VectorSubcoreMesh(core_axis_name="c", subcore_axis_name="s")

    @functools.partial(
        pl.kernel,
        mesh=mesh,
        out_type=jax.ShapeDtypeStruct((n_rows, _D), jnp.float32),
        scratch_types=[
            pltpu.VMEM((chunk,), jnp.int32),
            pltpu.VMEM((chunk, _D), jnp.float32),
            pltpu.SemaphoreType.DMA,
        ],
        compiler_params=pltpu.CompilerParams(use_tc_tiling_on_sc=False),
    )
    def gather(table_hbm, idx_hbm, out_hbm, idx_v, rows_v, sem):
        wid = lax.axis_index("s") * _NC + lax.axis_index("c")
        base = wid * b_per_w

        def body(i, carry):
            off = base + i * chunk
            pltpu.sync_copy(idx_hbm.at[pl.ds(off, chunk)], idx_v)
            pltpu.async_copy(table_hbm.at[idx_v], rows_v, sem).wait()
            pltpu.sync_copy(rows_v, out_hbm.at[pl.ds(off, chunk)])
            return carry

        lax.fori_loop(0, n_chunks, body, 0)

    return gather


def kernel(coeffs, weight):
    table = _renorm_table(weight)
    b, p = coeffs.shape
    idx = coeffs.reshape(-1)
    out = _make_gather(b * p, 2000)(table, idx)
    return out.reshape(b, p, _D)
